# baseline (speedup 1.0000x reference)
import functools

import jax
import jax.numpy as jnp
from jax import lax
from jax.experimental import pallas as pl
from jax.experimental.pallas import tpu as pltpu
from jax.experimental.pallas import tpu_sc as plsc

B = 16384
E = 32
NC = 2
NS = 16
NW = NC * NS
BPW = B // NW
CH = 128
NCH = BPW // CH
L = 16


def _sc_body(user_hbm, item_hbm, ie_hbm, fq_hbm, bias_hbm, utab_hbm,
             itab_hbm, ftab_hbm, out_hbm,
             uidx, iidx, iev, fqv, fidx, urows, irows, fvals, outv, biasv,
             sem):
    wid = lax.axis_index("s") * NC + lax.axis_index("c")
    base = wid * BPW

    pltpu.sync_copy(user_hbm.at[wid], uidx)
    pltpu.sync_copy(item_hbm.at[wid], iidx)
    pltpu.sync_copy(ie_hbm.at[wid], iev)
    pltpu.sync_copy(fq_hbm.at[wid], fqv)
    pltpu.sync_copy(bias_hbm, biasv)

    copies = []
    for j in range(NCH):
        copies.append(pltpu.async_copy(
            utab_hbm.at[uidx.at[j]], urows.at[pl.ds(j * CH, CH)], sem))
        copies.append(pltpu.async_copy(
            itab_hbm.at[iidx.at[j]], irows.at[pl.ds(j * CH, CH)], sem))

    for j in range(NCH):
        for k in range(CH // L):
            s = pl.ds(k * L, L)
            fidx[j, s] = iev[j, s] * 1000 + fqv[j, s]
    for j in range(NCH):
        copies.append(pltpu.async_copy(
            ftab_hbm.at[fidx.at[j]], fvals.at[pl.ds(j * CH, CH)], sem))
    for c in copies:
        c.wait()

    iota = lax.broadcasted_iota(jnp.int32, (L,), 0)
    cols = [jnp.full((L,), e, dtype=jnp.int32) for e in range(E)]
    bias_vec = biasv[pl.ds(0, L)]

    def dot_step(step, carry):
        r0 = step * L
        rows16 = iota + r0
        acc = [None, None, None, None]
        for e in range(E):
            ue = plsc.load_gather(urows, [rows16, cols[e]])
            ve = plsc.load_gather(irows, [rows16, cols[e]])
            p = ue * ve
            a = e % 4
            acc[a] = p if e < 4 else acc[a] + p
        tot = (acc[0] + acc[1]) + (acc[2] + acc[3])
        s = pl.ds(r0, L)
        outv[s] = tot + fvals[s] + bias_vec
        return carry

    lax.fori_loop(0, BPW // L, dot_step, 0)

    pltpu.sync_copy(outv, out_hbm.at[pl.ds(base, BPW)])


@jax.jit
def _sfc_forward(user, item, idx_emb, freq, bias, utab, itab, ftab):
    mesh = plsc.VectorSubcoreMesh(core_axis_name="c", subcore_axis_name="s")
    fwd = functools.partial(
        pl.kernel,
        mesh=mesh,
        compiler_params=pltpu.CompilerParams(
            use_tc_tiling_on_sc=False, needs_layout_passes=False),
        out_type=jax.ShapeDtypeStruct((B,), jnp.float32),
        scratch_types=[
            pltpu.VMEM((NCH, CH), jnp.int32),
            pltpu.VMEM((NCH, CH), jnp.int32),
            pltpu.VMEM((NCH, CH), jnp.int32),
            pltpu.VMEM((NCH, CH), jnp.int32),
            pltpu.VMEM((NCH, CH), jnp.int32),
            pltpu.VMEM((BPW, E), jnp.float32),
            pltpu.VMEM((BPW, E), jnp.float32),
            pltpu.VMEM((BPW,), jnp.float32),
            pltpu.VMEM((BPW,), jnp.float32),
            pltpu.VMEM((L,), jnp.float32),
            pltpu.SemaphoreType.DMA,
        ],
    )(_sc_body)
    return fwd(user, item, idx_emb, freq, bias, utab, itab, ftab)


def kernel(user, item, freq, idx_emb, zero, bias_table, user_table,
           item_table, freq_tables):
    del zero
    user3 = user.astype(jnp.int32).reshape(NW, NCH, CH)
    item3 = item.astype(jnp.int32).reshape(NW, NCH, CH)
    ie3 = idx_emb.astype(jnp.int32).reshape(NW, NCH, CH)
    fq3 = freq.astype(jnp.int32).reshape(NW, NCH, CH)
    bias1 = jnp.broadcast_to(bias_table.reshape(()), (L,))
    ftab_flat = freq_tables.reshape(-1)
    return _sfc_forward(user3, item3, ie3, fq3, bias1, user_table,
                        item_table, ftab_flat)
